# baseline (speedup 1.0000x reference)
import functools

import jax
import jax.numpy as jnp
import numpy as np
from jax import lax
from jax.experimental import pallas as pl
from jax.experimental.pallas import tpu as pltpu
from jax.experimental.pallas import tpu_sc as plsc

_B = 16
_C = 96
_HW = 4096
_NS = 16
_NG = 528
_U = 8
_GRP = 6


def _polar_constants():
    i = np.arange(32, dtype=np.float64)
    j = np.arange(64, dtype=np.float64)
    xs = (i - 16.0) / 16.0
    ys = (j - 32.0) / 32.0
    xg = np.broadcast_to(xs[:, None], (32, 64))
    yg = np.broadcast_to(ys[None, :], (32, 64))
    with np.errstate(divide="ignore"):
        radial = np.log(np.sqrt(xg ** 2 + yg ** 2))
    radial = np.maximum(radial, -30.0)
    a = np.arctan2(yg, xg)
    a = np.where(a > 0, a, 2.0 * np.pi + a)
    t = 0.5 * a * 64.0 / np.pi
    tc = t - 0.5
    def shuf(m):
        s = m.reshape(16, 2, 32, 2).transpose(1, 3, 0, 2).reshape(4, 512)
        return np.pad(s, ((0, 0), (0, _NG - 512))).astype(np.float32)
    return shuf(radial), shuf(tc)


_RADIAL_K, _TC_K = _polar_constants()


def _sc_sums_body(x_hbm, out_hbm, plane_v, sums_v, sem0, sem1):
    wid = lax.axis_index("s") * 2 + lax.axis_index("c")
    b = wid // 2
    c0 = (wid % 2) * (_GRP * _U)

    blk = _U * _HW
    sems = [sem0, sem1]
    copies = [None, None]
    copies[0] = pltpu.make_async_copy(
        x_hbm.at[b, pl.ds(c0 * _HW, blk)], plane_v.at[pl.ds(0, blk)], sems[0])
    copies[0].start()
    for g in range(_GRP):
        buf = g % 2
        copies[buf].wait()
        if g + 1 < _GRP:
            nbuf = 1 - buf
            copies[nbuf] = pltpu.make_async_copy(
                x_hbm.at[b, pl.ds((c0 + (g + 1) * _U) * _HW, blk)],
                plane_v.at[pl.ds(nbuf * blk, blk)], sems[nbuf])
            copies[nbuf].start()

        for u in range(_U):
            base = buf * blk + u * _HW
            def row_body(r, acc, base=base):
                off = base + r * 64
                for l in range(4):
                    acc = acc + plane_v[pl.ds(off + l * 16, 16)]
                return acc
            acc = lax.fori_loop(0, 64, row_body, jnp.zeros((16,), jnp.float32))
            sums_v[g * _U + u, :] = acc
    pltpu.sync_copy(sums_v, out_hbm.at[b, pl.ds(c0, _GRP * _U)])


def _sc_sums(x2):
    f = pl.kernel(
        _sc_sums_body,
        out_type=jax.ShapeDtypeStruct((_B, _C, 16), jnp.float32),
        mesh=plsc.VectorSubcoreMesh(core_axis_name="c", subcore_axis_name="s"),
        scratch_types=[
            pltpu.VMEM((2 * _U * _HW,), jnp.float32),
            pltpu.VMEM((_GRP * _U, 16), jnp.float32),
            pltpu.SemaphoreType.DMA,
            pltpu.SemaphoreType.DMA,
        ],
        compiler_params=pltpu.CompilerParams(needs_layout_passes=False),
    )
    return f(x2)


def _tc_prep_body(sums_ref, lt_ref, f1wt_ref, f1b_ref, f2wt_ref, f2b_ref,
                  rad_ref, tck_ref, idx_ref, w_ref):
    branch = jnp.sum(sums_ref[...], axis=2) * (1.0 / _HW)
    h = jnp.maximum(
        jnp.dot(branch, f1wt_ref[...],
                preferred_element_type=jnp.float32) + f1b_ref[...], 0.0)
    z = jnp.dot(h, f2wt_ref[...],
                preferred_element_type=jnp.float32) + f2b_ref[...]
    weight = 1.0 / (1.0 + jnp.exp(-z))
    w_ref[...] = weight

    w0 = weight[:, 0:1, None]
    w1 = weight[:, 1:2, None]
    lw0 = jnp.log(w0 * 0.01)
    lw1 = jnp.log(w1 * 0.6)
    dd = lw1 - lw0
    lx = lt_ref[...][:, 0:1, None]
    ly = lt_ref[...][:, 1:2, None]
    s_y = 64.0 / dd
    t_y = (-64.0) * lw0 / dd + (32.0 * ly - 0.5)

    iy = jnp.clip(rad_ref[...][None] * s_y + t_y, 0.0, 63.0)
    ix = jnp.clip(tck_ref[...][None] + 32.0 * lx, 0.0, 63.0)
    y0f = jnp.floor(iy)
    x0f = jnp.floor(ix)
    wy = iy - y0f
    wx = ix - x0f
    y0 = y0f.astype(jnp.int32)
    x0 = x0f.astype(jnp.int32)
    y1 = jnp.minimum(y0 + 1, 63)
    x1 = jnp.minimum(x0 + 1, 63)
    i00 = y0 * 64 + x0
    i01 = y0 * 64 + x1
    i10 = y1 * 64 + x0
    i11 = y1 * 64 + x1
    q = 0.25
    w00 = (1.0 - wy) * (1.0 - wx) * q
    w01 = (1.0 - wy) * wx * q
    w10 = wy * (1.0 - wx) * q
    w11 = wy * wx * q
    idx16 = jnp.concatenate([i00, i01, i10, i11], axis=1)
    wgt16 = jnp.concatenate([w00, w01, w10, w11], axis=1)

    cy = jnp.clip(32.0 * ly + 31.5, 0.0, 63.0)
    cx = jnp.clip(32.0 * lx + 31.5, 0.0, 63.0)
    cy0f = jnp.floor(cy)
    cx0f = jnp.floor(cx)
    cwy = cy - cy0f
    cwx = cx - cx0f
    cy0 = cy0f.astype(jnp.int32)
    cx0 = cx0f.astype(jnp.int32)
    cy1 = jnp.minimum(cy0 + 1, 63)
    cx1 = jnp.minimum(cx0 + 1, 63)
    row = lax.broadcasted_iota(jnp.int32, (_B, _NS, _NG), 1)
    col = lax.broadcasted_iota(jnp.int32, (_B, _NS, _NG), 2)
    cidx = jnp.where(row == 0, cy0 * 64 + cx0,
           jnp.where(row == 1, cy0 * 64 + cx1,
           jnp.where(row == 2, cy1 * 64 + cx0,
           jnp.where(row == 3, cy1 * 64 + cx1, 0))))
    cwgt = jnp.where(row == 0, (1.0 - cwy) * (1.0 - cwx),
           jnp.where(row == 1, (1.0 - cwy) * cwx,
           jnp.where(row == 2, cwy * (1.0 - cwx),
           jnp.where(row == 3, cwy * cwx, 0.0))))
    idx_f = jnp.where(col < 512, idx16, jnp.where(col == 512, cidx, 0))
    wgt_f = jnp.where(col < 512, wgt16, jnp.where(col == 512, cwgt, 0.0))
    idx_f = jnp.clip(idx_f, 0, _HW - 1)
    wb = lax.bitcast_convert_type(wgt_f, jnp.int32)
    wb = (wb + 0x7FFF + (lax.shift_right_logical(wb, 16) & 1)) & ~0xFFFF
    idx_ref[...] = wb | idx_f


def _tc_prep(sums, lt, f1wt, f1b, f2wt, f2b, interpret=False):
    rad = jnp.asarray(_RADIAL_K)
    tck = jnp.asarray(_TC_K)
    return pl.pallas_call(
        _tc_prep_body,
        out_shape=[
            jax.ShapeDtypeStruct((_B, _NS, _NG), jnp.int32),
            jax.ShapeDtypeStruct((_B, 2), jnp.float32),
        ],
        interpret=interpret,
    )(sums, lt, f1wt, f1b, f2wt, f2b, rad, tck)


def _sc_gather_body(x_hbm, idx_hbm, out_hbm,
                    idx_v, plane_v, out_v, sem0, sem1):
    wid = lax.axis_index("s") * 2 + lax.axis_index("c")
    b = wid // 2
    c0 = (wid % 2) * (_GRP * _U)

    pltpu.sync_copy(idx_hbm.at[b], idx_v)

    blk = _U * _HW
    sems = [sem0, sem1]
    copies = [None, None]
    copies[0] = pltpu.make_async_copy(
        x_hbm.at[b, pl.ds(c0 * _HW, blk)], plane_v.at[pl.ds(0, blk)], sems[0])
    copies[0].start()
    for g in range(_GRP):
        buf = g % 2
        copies[buf].wait()
        if g + 1 < _GRP:
            nbuf = 1 - buf
            copies[nbuf] = pltpu.make_async_copy(
                x_hbm.at[b, pl.ds((c0 + (g + 1) * _U) * _HW, blk)],
                plane_v.at[pl.ds(nbuf * blk, blk)], sems[nbuf])
            copies[nbuf].start()

        def chunk_body(i, carry, buf=buf):
            base = i * 16
            accs = [jnp.zeros((16,), jnp.float32) for _ in range(_U)]
            for s in range(_NS):
                iv = idx_v[s, pl.ds(base, 16)]
                wv = lax.bitcast_convert_type(iv & ~0xFFFF, jnp.float32)
                pv = lax.bitwise_and(iv, 0xFFFF)
                for u in range(_U):
                    gv = plsc.load_gather(
                        plane_v, [pv + (buf * blk + u * _HW)])
                    accs[u] = accs[u] + wv * gv
            for u in range(_U):
                out_v[u, pl.ds(base, 16)] = accs[u]
            return carry

        lax.fori_loop(0, _NG // 16, chunk_body, 0)
        pltpu.sync_copy(out_v, out_hbm.at[b, pl.ds(c0 + g * _U, _U)])


def _sc_gather(x2, idx):
    f = pl.kernel(
        _sc_gather_body,
        out_type=jax.ShapeDtypeStruct((_B, _C, _NG), jnp.float32),
        mesh=plsc.VectorSubcoreMesh(core_axis_name="c", subcore_axis_name="s"),
        scratch_types=[
            pltpu.VMEM((_NS, _NG), jnp.int32),
            pltpu.VMEM((2 * _U * _HW,), jnp.float32),
            pltpu.VMEM((_U, _NG), jnp.float32),
            pltpu.SemaphoreType.DMA,
            pltpu.SemaphoreType.DMA,
        ],
        compiler_params=pltpu.CompilerParams(needs_layout_passes=False),
    )
    return f(x2, idx)


def kernel(x, l_t_prev, fc1_w, fc1_b, fc2_w, fc2_b):
    x2 = x.reshape(_B, _C * _HW)
    sums = _sc_sums(x2)
    idx, weight = _tc_prep(
        sums, l_t_prev, fc1_w.T, fc1_b.reshape(1, 48),
        fc2_w.T, fc2_b.reshape(1, 2))
    p = _sc_gather(x2, idx)
    fov = p[:, :, :512].reshape(_B, _C, 16, 32)
    vconst = p[:, :, 512]
    pooled = jnp.broadcast_to(vconst[:, :, None, None], (_B, _C, 32, 64))
    pooled = pooled.at[:, :, :16, :32].set(fov)
    return pooled, weight


# speedup vs baseline: 1.0853x; 1.0853x over previous
import functools

import jax
import jax.numpy as jnp
import numpy as np
from jax import lax
from jax.experimental import pallas as pl
from jax.experimental.pallas import tpu as pltpu
from jax.experimental.pallas import tpu_sc as plsc

_B = 16
_C = 96
_HW = 4096
_NS = 16
_NG = 528
_U = 8
_GRP = 6


def _polar_constants():
    i = np.arange(32, dtype=np.float64)
    j = np.arange(64, dtype=np.float64)
    xs = (i - 16.0) / 16.0
    ys = (j - 32.0) / 32.0
    xg = np.broadcast_to(xs[:, None], (32, 64))
    yg = np.broadcast_to(ys[None, :], (32, 64))
    with np.errstate(divide="ignore"):
        radial = np.log(np.sqrt(xg ** 2 + yg ** 2))
    radial = np.maximum(radial, -30.0)
    a = np.arctan2(yg, xg)
    a = np.where(a > 0, a, 2.0 * np.pi + a)
    t = 0.5 * a * 64.0 / np.pi
    tc = t - 0.5
    def shuf(m):
        s = m.reshape(16, 2, 32, 2).transpose(1, 3, 0, 2).reshape(4, 512)
        return np.pad(s, ((0, 0), (0, _NG - 512))).astype(np.float32)
    return shuf(radial), shuf(tc)


_RADIAL_K, _TC_K = _polar_constants()


def _tc_prep_body(lt_ref, x_ref, f1w_ref, f1b_ref, f2w_ref, f2b_ref,
                  rad_ref, tck_ref, idx_ref, w_ref):
    b = pl.program_id(0)
    xb = x_ref[0].reshape(_C, 32, 128)
    branch = (jnp.sum(xb, axis=(1, 2)) * (1.0 / _HW)).reshape(_C, 1)
    h = jnp.maximum(jnp.dot(f1w_ref[...], branch,
                            preferred_element_type=jnp.float32) + f1b_ref[...], 0.0)
    z = jnp.dot(f2w_ref[...], h, preferred_element_type=jnp.float32) + f2b_ref[...]
    wcol = 1.0 / (1.0 + jnp.exp(-z))
    w_ref[...] = wcol[None]

    w0 = wcol[0:1, 0:1]
    w1 = wcol[1:2, 0:1]
    lw0 = jnp.log(w0 * 0.01)
    lw1 = jnp.log(w1 * 0.6)
    dd = lw1 - lw0
    lx = lt_ref[b, 0]
    ly = lt_ref[b, 1]
    s_y = 64.0 / dd
    t_y = (-64.0) * lw0 / dd + (32.0 * ly - 0.5)

    iy = jnp.clip(rad_ref[...] * s_y + t_y, 0.0, 63.0)
    ix = jnp.clip(tck_ref[...] + 32.0 * lx, 0.0, 63.0)
    y0f = jnp.floor(iy)
    x0f = jnp.floor(ix)
    wy = iy - y0f
    wx = ix - x0f
    y0 = y0f.astype(jnp.int32)
    x0 = x0f.astype(jnp.int32)
    y1 = jnp.minimum(y0 + 1, 63)
    x1 = jnp.minimum(x0 + 1, 63)
    i00 = y0 * 64 + x0
    i01 = y0 * 64 + x1
    i10 = y1 * 64 + x0
    i11 = y1 * 64 + x1
    q = 0.25
    w00 = (1.0 - wy) * (1.0 - wx) * q
    w01 = (1.0 - wy) * wx * q
    w10 = wy * (1.0 - wx) * q
    w11 = wy * wx * q
    idx16 = jnp.concatenate([i00, i01, i10, i11], axis=0)
    wgt16 = jnp.concatenate([w00, w01, w10, w11], axis=0)

    cy = jnp.clip(32.0 * ly + 31.5, 0.0, 63.0)
    cx = jnp.clip(32.0 * lx + 31.5, 0.0, 63.0)
    cy0f = jnp.floor(cy)
    cx0f = jnp.floor(cx)
    cwy = cy - cy0f
    cwx = cx - cx0f
    cy0 = cy0f.astype(jnp.int32)
    cx0 = cx0f.astype(jnp.int32)
    cy1 = jnp.minimum(cy0 + 1, 63)
    cx1 = jnp.minimum(cx0 + 1, 63)
    row = lax.broadcasted_iota(jnp.int32, (_NS, _NG), 0)
    col = lax.broadcasted_iota(jnp.int32, (_NS, _NG), 1)
    cidx = jnp.where(row == 0, cy0 * 64 + cx0,
           jnp.where(row == 1, cy0 * 64 + cx1,
           jnp.where(row == 2, cy1 * 64 + cx0,
           jnp.where(row == 3, cy1 * 64 + cx1, 0))))
    cwgt = jnp.where(row == 0, (1.0 - cwy) * (1.0 - cwx),
           jnp.where(row == 1, (1.0 - cwy) * cwx,
           jnp.where(row == 2, cwy * (1.0 - cwx),
           jnp.where(row == 3, cwy * cwx, 0.0))))
    idx_f = jnp.where(col < 512, idx16, jnp.where(col == 512, cidx, 0))
    wgt_f = jnp.where(col < 512, wgt16, jnp.where(col == 512, cwgt, 0.0))
    idx_f = jnp.clip(idx_f, 0, _HW - 1)
    wb = lax.bitcast_convert_type(wgt_f, jnp.int32)
    wb = (wb + 0x7FFF + (lax.shift_right_logical(wb, 16) & 1)) & ~0xFFFF
    idx_ref[0] = wb | idx_f


def _tc_prep(xd, lt, f1w, f1b, f2w, f2b, interpret=False):
    rad = jnp.asarray(_RADIAL_K)
    tck = jnp.asarray(_TC_K)
    return pl.pallas_call(
        _tc_prep_body,
        grid=(_B,),
        in_specs=[
            pl.BlockSpec(memory_space=pltpu.SMEM),
            pl.BlockSpec((1, _C * 32, 128), lambda b: (b, 0, 0)),
            pl.BlockSpec((48, _C), lambda b: (0, 0)),
            pl.BlockSpec((48, 1), lambda b: (0, 0)),
            pl.BlockSpec((2, 48), lambda b: (0, 0)),
            pl.BlockSpec((2, 1), lambda b: (0, 0)),
            pl.BlockSpec((4, _NG), lambda b: (0, 0)),
            pl.BlockSpec((4, _NG), lambda b: (0, 0)),
        ],
        out_specs=[
            pl.BlockSpec((1, _NS, _NG), lambda b: (b, 0, 0)),
            pl.BlockSpec((1, 2, 1), lambda b: (b, 0, 0)),
        ],
        out_shape=[
            jax.ShapeDtypeStruct((_B, _NS, _NG), jnp.int32),
            jax.ShapeDtypeStruct((_B, 2, 1), jnp.float32),
        ],
        interpret=interpret,
    )(lt, xd, f1w, f1b, f2w, f2b, rad, tck)


def _sc_gather_body(x_hbm, idx_hbm, out_hbm,
                    idx_v, plane_v, out_v, sem0, sem1):
    wid = lax.axis_index("s") * 2 + lax.axis_index("c")
    b = wid // 2
    c0 = (wid % 2) * (_GRP * _U)

    pltpu.sync_copy(idx_hbm.at[b], idx_v)

    rblk = _U * 32
    sems = [sem0, sem1]
    copies = [None, None]
    copies[0] = pltpu.make_async_copy(
        x_hbm.at[b, pl.ds(c0 * 32, rblk)], plane_v.at[pl.ds(0, rblk)], sems[0])
    copies[0].start()
    for g in range(_GRP):
        buf = g % 2
        copies[buf].wait()
        if g + 1 < _GRP:
            nbuf = 1 - buf
            copies[nbuf] = pltpu.make_async_copy(
                x_hbm.at[b, pl.ds((c0 + (g + 1) * _U) * 32, rblk)],
                plane_v.at[pl.ds(nbuf * rblk, rblk)], sems[nbuf])
            copies[nbuf].start()

        planes = [plane_v.at[pl.ds((buf * _U + u) * 32, 32), :]
                  for u in range(_U)]

        def chunk_body(i, carry, planes=planes):
            base = i * 16
            accs = [jnp.zeros((16,), jnp.float32) for _ in range(_U)]
            for s in range(_NS):
                iv = idx_v[s, pl.ds(base, 16)]
                wv = lax.bitcast_convert_type(iv & ~0xFFFF, jnp.float32)
                pv = lax.bitwise_and(iv, 0xFFFF)
                rv = lax.shift_right_logical(pv, 7)
                cv = lax.bitwise_and(pv, 127)
                for u in range(_U):
                    gv = plsc.load_gather(planes[u], [rv, cv])
                    accs[u] = accs[u] + wv * gv
            for u in range(_U):
                out_v[u, pl.ds(base, 16)] = accs[u]
            return carry

        lax.fori_loop(0, _NG // 16, chunk_body, 0)
        pltpu.sync_copy(out_v, out_hbm.at[b, pl.ds(c0 + g * _U, _U)])


def _sc_gather(xd, idx):
    f = pl.kernel(
        _sc_gather_body,
        out_type=jax.ShapeDtypeStruct((_B, _C, _NG), jnp.float32),
        mesh=plsc.VectorSubcoreMesh(core_axis_name="c", subcore_axis_name="s"),
        scratch_types=[
            pltpu.VMEM((_NS, _NG), jnp.int32),
            pltpu.VMEM((2 * _U * 32, 128), jnp.float32),
            pltpu.VMEM((_U, _NG), jnp.float32),
            pltpu.SemaphoreType.DMA,
            pltpu.SemaphoreType.DMA,
        ],
        compiler_params=pltpu.CompilerParams(needs_layout_passes=False),
    )
    return f(xd, idx)


def kernel(x, l_t_prev, fc1_w, fc1_b, fc2_w, fc2_b):
    xd = x.reshape(_B, _C * 32, 128)
    idx, wout = _tc_prep(
        xd, l_t_prev, fc1_w, fc1_b.reshape(48, 1), fc2_w, fc2_b.reshape(2, 1))
    weight = wout[:, :, 0]
    p = _sc_gather(xd, idx)
    fov = p[:, :, :512].reshape(_B, _C, 16, 32)
    vconst = p[:, :, 512]
    pooled = jnp.broadcast_to(vconst[:, :, None, None], (_B, _C, 32, 64))
    pooled = pooled.at[:, :, :16, :32].set(fov)
    return pooled, weight


# speedup vs baseline: 1.1246x; 1.0361x over previous
import functools

import jax
import jax.numpy as jnp
import numpy as np
from jax import lax
from jax.experimental import pallas as pl
from jax.experimental.pallas import tpu as pltpu
from jax.experimental.pallas import tpu_sc as plsc

_B = 16
_C = 96
_HW = 4096
_NS = 16
_NG = 528
_U = 8
_GRP = 6


def _polar_constants():
    i = np.arange(32, dtype=np.float64)
    j = np.arange(64, dtype=np.float64)
    xs = (i - 16.0) / 16.0
    ys = (j - 32.0) / 32.0
    xg = np.broadcast_to(xs[:, None], (32, 64))
    yg = np.broadcast_to(ys[None, :], (32, 64))
    with np.errstate(divide="ignore"):
        radial = np.log(np.sqrt(xg ** 2 + yg ** 2))
    radial = np.maximum(radial, -30.0)
    a = np.arctan2(yg, xg)
    a = np.where(a > 0, a, 2.0 * np.pi + a)
    t = 0.5 * a * 64.0 / np.pi
    tc = t - 0.5
    def shuf(m):
        s = m.reshape(16, 2, 32, 2).transpose(1, 3, 0, 2).reshape(4, 512)
        return np.pad(s, ((0, 0), (0, _NG - 512))).astype(np.float32)
    return shuf(radial), shuf(tc)


_RADIAL_K, _TC_K = _polar_constants()


def _tc_prep_sub(b, xb, lt_ref, f1w_ref, f1b_ref, f2w_ref, f2b_ref,
                 rad_ref, tck_ref):
    branch = (jnp.sum(xb, axis=(1, 2)) * (1.0 / _HW)).reshape(_C, 1)
    h = jnp.maximum(jnp.dot(f1w_ref[...], branch,
                            preferred_element_type=jnp.float32) + f1b_ref[...], 0.0)
    z = jnp.dot(f2w_ref[...], h, preferred_element_type=jnp.float32) + f2b_ref[...]
    wcol = 1.0 / (1.0 + jnp.exp(-z))

    w0 = wcol[0:1, 0:1]
    w1 = wcol[1:2, 0:1]
    lw0 = jnp.log(w0 * 0.01)
    lw1 = jnp.log(w1 * 0.6)
    dd = lw1 - lw0
    lx = lt_ref[b, 0]
    ly = lt_ref[b, 1]
    s_y = 64.0 / dd
    t_y = (-64.0) * lw0 / dd + (32.0 * ly - 0.5)

    iy = jnp.clip(rad_ref[...] * s_y + t_y, 0.0, 63.0)
    ix = jnp.clip(tck_ref[...] + 32.0 * lx, 0.0, 63.0)
    y0f = jnp.floor(iy)
    x0f = jnp.floor(ix)
    wy = iy - y0f
    wx = ix - x0f
    y0 = y0f.astype(jnp.int32)
    x0 = x0f.astype(jnp.int32)
    y1 = jnp.minimum(y0 + 1, 63)
    x1 = jnp.minimum(x0 + 1, 63)
    i00 = y0 * 64 + x0
    i01 = y0 * 64 + x1
    i10 = y1 * 64 + x0
    i11 = y1 * 64 + x1
    q = 0.25
    w00 = (1.0 - wy) * (1.0 - wx) * q
    w01 = (1.0 - wy) * wx * q
    w10 = wy * (1.0 - wx) * q
    w11 = wy * wx * q
    idx16 = jnp.concatenate([i00, i01, i10, i11], axis=0)
    wgt16 = jnp.concatenate([w00, w01, w10, w11], axis=0)

    cy = jnp.clip(32.0 * ly + 31.5, 0.0, 63.0)
    cx = jnp.clip(32.0 * lx + 31.5, 0.0, 63.0)
    cy0f = jnp.floor(cy)
    cx0f = jnp.floor(cx)
    cwy = cy - cy0f
    cwx = cx - cx0f
    cy0 = cy0f.astype(jnp.int32)
    cx0 = cx0f.astype(jnp.int32)
    cy1 = jnp.minimum(cy0 + 1, 63)
    cx1 = jnp.minimum(cx0 + 1, 63)
    row = lax.broadcasted_iota(jnp.int32, (_NS, _NG), 0)
    col = lax.broadcasted_iota(jnp.int32, (_NS, _NG), 1)
    cidx = jnp.where(row == 0, cy0 * 64 + cx0,
           jnp.where(row == 1, cy0 * 64 + cx1,
           jnp.where(row == 2, cy1 * 64 + cx0,
           jnp.where(row == 3, cy1 * 64 + cx1, 0))))
    cwgt = jnp.where(row == 0, (1.0 - cwy) * (1.0 - cwx),
           jnp.where(row == 1, (1.0 - cwy) * cwx,
           jnp.where(row == 2, cwy * (1.0 - cwx),
           jnp.where(row == 3, cwy * cwx, 0.0))))
    idx_f = jnp.where(col < 512, idx16, jnp.where(col == 512, cidx, 0))
    wgt_f = jnp.where(col < 512, wgt16, jnp.where(col == 512, cwgt, 0.0))
    idx_f = jnp.clip(idx_f, 0, _HW - 1)
    wb = lax.bitcast_convert_type(wgt_f, jnp.int32)
    wb = (wb + 0x7FFF + (lax.shift_right_logical(wb, 16) & 1)) & ~0xFFFF
    return wb | idx_f, wcol


def _tc_prep_body(lt_ref, x_ref, f1w_ref, f1b_ref, f2w_ref, f2b_ref,
                  rad_ref, tck_ref, idx_ref, w_ref):
    pb = pl.program_id(0)
    for k in range(2):
        xb = x_ref[k].reshape(_C, 32, 128)
        packed, wcol = _tc_prep_sub(pb * 2 + k, xb, lt_ref, f1w_ref, f1b_ref,
                                    f2w_ref, f2b_ref, rad_ref, tck_ref)
        idx_ref[k] = packed
        w_ref[k] = wcol


def _tc_prep(xd, lt, f1w, f1b, f2w, f2b, interpret=False):
    rad = jnp.asarray(_RADIAL_K)
    tck = jnp.asarray(_TC_K)
    return pl.pallas_call(
        _tc_prep_body,
        grid=(_B // 2,),
        in_specs=[
            pl.BlockSpec(memory_space=pltpu.SMEM),
            pl.BlockSpec((2, _C * 32, 128), lambda b: (b, 0, 0)),
            pl.BlockSpec((48, _C), lambda b: (0, 0)),
            pl.BlockSpec((48, 1), lambda b: (0, 0)),
            pl.BlockSpec((2, 48), lambda b: (0, 0)),
            pl.BlockSpec((2, 1), lambda b: (0, 0)),
            pl.BlockSpec((4, _NG), lambda b: (0, 0)),
            pl.BlockSpec((4, _NG), lambda b: (0, 0)),
        ],
        out_specs=[
            pl.BlockSpec((2, _NS, _NG), lambda b: (b, 0, 0)),
            pl.BlockSpec((2, 2, 1), lambda b: (b, 0, 0)),
        ],
        out_shape=[
            jax.ShapeDtypeStruct((_B, _NS, _NG), jnp.int32),
            jax.ShapeDtypeStruct((_B, 2, 1), jnp.float32),
        ],
        interpret=interpret,
    )(lt, xd, f1w, f1b, f2w, f2b, rad, tck)


def _sc_gather_body(x_hbm, idx_hbm, out_hbm,
                    idx_v, plane_v, out_v, sem0, sem1):
    wid = lax.axis_index("s") * 2 + lax.axis_index("c")
    b = wid // 2
    c0 = (wid % 2) * (_GRP * _U)

    pltpu.sync_copy(idx_hbm.at[b], idx_v)

    rblk = _U * 32
    sems = [sem0, sem1]
    copies = [None, None]
    copies[0] = pltpu.make_async_copy(
        x_hbm.at[b, pl.ds(c0 * 32, rblk)], plane_v.at[pl.ds(0, rblk)], sems[0])
    copies[0].start()
    for g in range(_GRP):
        buf = g % 2
        copies[buf].wait()
        if g + 1 < _GRP:
            nbuf = 1 - buf
            copies[nbuf] = pltpu.make_async_copy(
                x_hbm.at[b, pl.ds((c0 + (g + 1) * _U) * 32, rblk)],
                plane_v.at[pl.ds(nbuf * rblk, rblk)], sems[nbuf])
            copies[nbuf].start()

        planes = [plane_v.at[pl.ds((buf * _U + u) * 32, 32), :]
                  for u in range(_U)]

        def chunk_body(i, carry, planes=planes):
            base = i * 16
            accs = [jnp.zeros((16,), jnp.float32) for _ in range(_U)]
            for s in range(_NS):
                iv = idx_v[s, pl.ds(base, 16)]
                wv = lax.bitcast_convert_type(iv & ~0xFFFF, jnp.float32)
                pv = lax.bitwise_and(iv, 0xFFFF)
                rv = lax.shift_right_logical(pv, 7)
                cv = lax.bitwise_and(pv, 127)
                for u in range(_U):
                    gv = plsc.load_gather(planes[u], [rv, cv])
                    accs[u] = accs[u] + wv * gv
            for u in range(_U):
                out_v[u, pl.ds(base, 16)] = accs[u]
            return carry

        lax.fori_loop(0, _NG // 16, chunk_body, 0)
        pltpu.sync_copy(out_v, out_hbm.at[b, pl.ds(c0 + g * _U, _U)])


def _sc_gather(xd, idx):
    f = pl.kernel(
        _sc_gather_body,
        out_type=jax.ShapeDtypeStruct((_B, _C, _NG), jnp.float32),
        mesh=plsc.VectorSubcoreMesh(core_axis_name="c", subcore_axis_name="s"),
        scratch_types=[
            pltpu.VMEM((_NS, _NG), jnp.int32),
            pltpu.VMEM((2 * _U * 32, 128), jnp.float32),
            pltpu.VMEM((_U, _NG), jnp.float32),
            pltpu.SemaphoreType.DMA,
            pltpu.SemaphoreType.DMA,
        ],
        compiler_params=pltpu.CompilerParams(needs_layout_passes=False),
    )
    return f(xd, idx)


def kernel(x, l_t_prev, fc1_w, fc1_b, fc2_w, fc2_b):
    xd = x.reshape(_B, _C * 32, 128)
    idx, wout = _tc_prep(
        xd, l_t_prev, fc1_w, fc1_b.reshape(48, 1), fc2_w, fc2_b.reshape(2, 1))
    weight = wout[:, :, 0]
    p = _sc_gather(xd, idx)
    fov = p[:, :, :512].reshape(_B, _C, 16, 32)
    vconst = p[:, :, 512]
    pooled = jnp.broadcast_to(vconst[:, :, None, None], (_B, _C, 32, 64))
    pooled = pooled.at[:, :, :16, :32].set(fov)
    return pooled, weight


# speedup vs baseline: 1.1408x; 1.0145x over previous
import functools

import jax
import jax.numpy as jnp
import numpy as np
from jax import lax
from jax.experimental import pallas as pl
from jax.experimental.pallas import tpu as pltpu
from jax.experimental.pallas import tpu_sc as plsc

_B = 16
_C = 96
_HW = 4096
_NS = 16
_NG = 528
_U = 8
_GRP = 6


def _polar_constants():
    i = np.arange(32, dtype=np.float64)
    j = np.arange(64, dtype=np.float64)
    xs = (i - 16.0) / 16.0
    ys = (j - 32.0) / 32.0
    xg = np.broadcast_to(xs[:, None], (32, 64))
    yg = np.broadcast_to(ys[None, :], (32, 64))
    with np.errstate(divide="ignore"):
        radial = np.log(np.sqrt(xg ** 2 + yg ** 2))
    radial = np.maximum(radial, -30.0)
    a = np.arctan2(yg, xg)
    a = np.where(a > 0, a, 2.0 * np.pi + a)
    t = 0.5 * a * 64.0 / np.pi
    tc = t - 0.5
    def shuf(m):
        s = m.reshape(16, 2, 32, 2).transpose(1, 3, 0, 2).reshape(4, 512)
        return np.pad(s, ((0, 0), (0, _NG - 512))).astype(np.float32)
    return shuf(radial), shuf(tc)


_RADIAL_K, _TC_K = _polar_constants()


def _tc_prep_sub(b, xb, lt_ref, f1w_ref, f1b_ref, f2w_ref, f2b_ref,
                 rad_ref, tck_ref):
    branch = (jnp.sum(xb, axis=(1, 2)) * (1.0 / _HW)).reshape(_C, 1)
    h = jnp.maximum(jnp.dot(f1w_ref[...], branch,
                            preferred_element_type=jnp.float32) + f1b_ref[...], 0.0)
    z = jnp.dot(f2w_ref[...], h, preferred_element_type=jnp.float32) + f2b_ref[...]
    wcol = 1.0 / (1.0 + jnp.exp(-z))

    w0 = wcol[0:1, 0:1]
    w1 = wcol[1:2, 0:1]
    lw0 = jnp.log(w0 * 0.01)
    lw1 = jnp.log(w1 * 0.6)
    dd = lw1 - lw0
    lx = lt_ref[b, 0]
    ly = lt_ref[b, 1]
    s_y = 64.0 / dd
    t_y = (-64.0) * lw0 / dd + (32.0 * ly - 0.5)

    iy = jnp.clip(rad_ref[...] * s_y + t_y, 0.0, 63.0)
    ix = jnp.clip(tck_ref[...] + 32.0 * lx, 0.0, 63.0)
    y0f = jnp.floor(iy)
    x0f = jnp.floor(ix)
    wy = iy - y0f
    wx = ix - x0f
    y0 = y0f.astype(jnp.int32)
    x0 = x0f.astype(jnp.int32)
    y1 = jnp.minimum(y0 + 1, 63)
    x1 = jnp.minimum(x0 + 1, 63)
    i00 = y0 * 64 + x0
    i01 = y0 * 64 + x1
    i10 = y1 * 64 + x0
    i11 = y1 * 64 + x1
    q = 0.25
    w00 = (1.0 - wy) * (1.0 - wx) * q
    w01 = (1.0 - wy) * wx * q
    w10 = wy * (1.0 - wx) * q
    w11 = wy * wx * q
    idx16 = jnp.concatenate([i00, i01, i10, i11], axis=0)
    wgt16 = jnp.concatenate([w00, w01, w10, w11], axis=0)

    cy = jnp.clip(32.0 * ly + 31.5, 0.0, 63.0)
    cx = jnp.clip(32.0 * lx + 31.5, 0.0, 63.0)
    cy0f = jnp.floor(cy)
    cx0f = jnp.floor(cx)
    cwy = cy - cy0f
    cwx = cx - cx0f
    cy0 = cy0f.astype(jnp.int32)
    cx0 = cx0f.astype(jnp.int32)
    cy1 = jnp.minimum(cy0 + 1, 63)
    cx1 = jnp.minimum(cx0 + 1, 63)
    row = lax.broadcasted_iota(jnp.int32, (_NS, _NG), 0)
    col = lax.broadcasted_iota(jnp.int32, (_NS, _NG), 1)
    cidx = jnp.where(row == 0, cy0 * 64 + cx0,
           jnp.where(row == 1, cy0 * 64 + cx1,
           jnp.where(row == 2, cy1 * 64 + cx0,
           jnp.where(row == 3, cy1 * 64 + cx1, 0))))
    cwgt = jnp.where(row == 0, (1.0 - cwy) * (1.0 - cwx),
           jnp.where(row == 1, (1.0 - cwy) * cwx,
           jnp.where(row == 2, cwy * (1.0 - cwx),
           jnp.where(row == 3, cwy * cwx, 0.0))))
    idx_f = jnp.where(col < 512, idx16, jnp.where(col == 512, cidx, 0))
    wgt_f = jnp.where(col < 512, wgt16, jnp.where(col == 512, cwgt, 0.0))
    idx_f = jnp.clip(idx_f, 0, _HW - 1)
    wb = lax.bitcast_convert_type(wgt_f, jnp.int32)
    wb = (wb + 0x7FFF + (lax.shift_right_logical(wb, 16) & 1)) & ~0xFFFF
    return wb | idx_f, wcol


def _tc_prep_body(lt_ref, x_ref, f1w_ref, f1b_ref, f2w_ref, f2b_ref,
                  rad_ref, tck_ref, idx_ref, w_ref):
    pb = pl.program_id(0)
    for k in range(4):
        xb = x_ref[k].reshape(_C, 32, 128)
        packed, wcol = _tc_prep_sub(pb * 4 + k, xb, lt_ref, f1w_ref, f1b_ref,
                                    f2w_ref, f2b_ref, rad_ref, tck_ref)
        idx_ref[k] = packed
        w_ref[k] = wcol


def _tc_prep(xd, lt, f1w, f1b, f2w, f2b, interpret=False):
    rad = jnp.asarray(_RADIAL_K)
    tck = jnp.asarray(_TC_K)
    return pl.pallas_call(
        _tc_prep_body,
        grid=(_B // 4,),
        in_specs=[
            pl.BlockSpec(memory_space=pltpu.SMEM),
            pl.BlockSpec((4, _C * 32, 128), lambda b: (b, 0, 0)),
            pl.BlockSpec((48, _C), lambda b: (0, 0)),
            pl.BlockSpec((48, 1), lambda b: (0, 0)),
            pl.BlockSpec((2, 48), lambda b: (0, 0)),
            pl.BlockSpec((2, 1), lambda b: (0, 0)),
            pl.BlockSpec((4, _NG), lambda b: (0, 0)),
            pl.BlockSpec((4, _NG), lambda b: (0, 0)),
        ],
        out_specs=[
            pl.BlockSpec((4, _NS, _NG), lambda b: (b, 0, 0)),
            pl.BlockSpec((4, 2, 1), lambda b: (b, 0, 0)),
        ],
        out_shape=[
            jax.ShapeDtypeStruct((_B, _NS, _NG), jnp.int32),
            jax.ShapeDtypeStruct((_B, 2, 1), jnp.float32),
        ],
        interpret=interpret,
    )(lt, xd, f1w, f1b, f2w, f2b, rad, tck)


def _sc_gather_body(x_hbm, idx_hbm, out_hbm,
                    idx_v, plane_v, out_v, sem0, sem1):
    wid = lax.axis_index("s") * 2 + lax.axis_index("c")
    b = wid // 2
    c0 = (wid % 2) * (_GRP * _U)

    pltpu.sync_copy(idx_hbm.at[b], idx_v)

    rblk = _U * 32
    sems = [sem0, sem1]
    copies = [None, None]
    copies[0] = pltpu.make_async_copy(
        x_hbm.at[b, pl.ds(c0 * 32, rblk)], plane_v.at[pl.ds(0, rblk)], sems[0])
    copies[0].start()
    for g in range(_GRP):
        buf = g % 2
        copies[buf].wait()
        if g + 1 < _GRP:
            nbuf = 1 - buf
            copies[nbuf] = pltpu.make_async_copy(
                x_hbm.at[b, pl.ds((c0 + (g + 1) * _U) * 32, rblk)],
                plane_v.at[pl.ds(nbuf * rblk, rblk)], sems[nbuf])
            copies[nbuf].start()

        planes = [plane_v.at[pl.ds((buf * _U + u) * 32, 32), :]
                  for u in range(_U)]

        def chunk_body(i, carry, planes=planes):
            base = i * 16
            accs = [jnp.zeros((16,), jnp.float32) for _ in range(_U)]
            for s in range(_NS):
                iv = idx_v[s, pl.ds(base, 16)]
                wv = lax.bitcast_convert_type(iv & ~0xFFFF, jnp.float32)
                pv = lax.bitwise_and(iv, 0xFFFF)
                rv = lax.shift_right_logical(pv, 7)
                cv = lax.bitwise_and(pv, 127)
                for u in range(_U):
                    gv = plsc.load_gather(planes[u], [rv, cv])
                    accs[u] = accs[u] + wv * gv
            for u in range(_U):
                out_v[u, pl.ds(base, 16)] = accs[u]
            return carry

        lax.fori_loop(0, _NG // 16, chunk_body, 0)
        pltpu.sync_copy(out_v, out_hbm.at[b, pl.ds(c0 + g * _U, _U)])


def _sc_gather(xd, idx):
    f = pl.kernel(
        _sc_gather_body,
        out_type=jax.ShapeDtypeStruct((_B, _C, _NG), jnp.float32),
        mesh=plsc.VectorSubcoreMesh(core_axis_name="c", subcore_axis_name="s"),
        scratch_types=[
            pltpu.VMEM((_NS, _NG), jnp.int32),
            pltpu.VMEM((2 * _U * 32, 128), jnp.float32),
            pltpu.VMEM((_U, _NG), jnp.float32),
            pltpu.SemaphoreType.DMA,
            pltpu.SemaphoreType.DMA,
        ],
        compiler_params=pltpu.CompilerParams(needs_layout_passes=False),
    )
    return f(xd, idx)


def kernel(x, l_t_prev, fc1_w, fc1_b, fc2_w, fc2_b):
    xd = x.reshape(_B, _C * 32, 128)
    idx, wout = _tc_prep(
        xd, l_t_prev, fc1_w, fc1_b.reshape(48, 1), fc2_w, fc2_b.reshape(2, 1))
    weight = wout[:, :, 0]
    p = _sc_gather(xd, idx)
    fov = p[:, :, :512].reshape(_B, _C, 16, 32)
    vconst = p[:, :, 512]
    pooled = jnp.broadcast_to(vconst[:, :, None, None], (_B, _C, 32, 64))
    pooled = pooled.at[:, :, :16, :32].set(fov)
    return pooled, weight


# speedup vs baseline: 1.1420x; 1.0010x over previous
import jax
import jax.numpy as jnp
import numpy as np
from jax import lax
from jax.experimental import pallas as pl
from jax.experimental.pallas import tpu as pltpu
from jax.experimental.pallas import tpu_sc as plsc

_B = 16
_C = 96
_HW = 4096
_NS = 16
_NG = 528
_U = 8
_GRP = 6


def _polar_constants():
    i = np.arange(32, dtype=np.float64)
    j = np.arange(64, dtype=np.float64)
    xs = (i - 16.0) / 16.0
    ys = (j - 32.0) / 32.0
    xg = np.broadcast_to(xs[:, None], (32, 64))
    yg = np.broadcast_to(ys[None, :], (32, 64))
    with np.errstate(divide="ignore"):
        radial = np.log(np.sqrt(xg ** 2 + yg ** 2))
    radial = np.maximum(radial, -30.0)
    a = np.arctan2(yg, xg)
    a = np.where(a > 0, a, 2.0 * np.pi + a)
    t = 0.5 * a * 64.0 / np.pi
    tc = t - 0.5
    def shuf(m):
        s = m.reshape(16, 2, 32, 2).transpose(1, 3, 0, 2).reshape(4, 512)
        return np.pad(s, ((0, 0), (0, _NG - 512))).astype(np.float32)
    return shuf(radial), shuf(tc)


_RADIAL_K, _TC_K = _polar_constants()


def _tc_prep_sub(b, xb, lt_ref, f1w_ref, f1b_ref, f2w_ref, f2b_ref,
                 rad_ref, tck_ref):
    branch = (jnp.sum(xb, axis=(1, 2)) * (1.0 / _HW)).reshape(_C, 1)
    h = jnp.maximum(jnp.dot(f1w_ref[...], branch,
                            preferred_element_type=jnp.float32) + f1b_ref[...], 0.0)
    z = jnp.dot(f2w_ref[...], h, preferred_element_type=jnp.float32) + f2b_ref[...]
    wcol = 1.0 / (1.0 + jnp.exp(-z))

    w0 = wcol[0:1, 0:1]
    w1 = wcol[1:2, 0:1]
    lw0 = jnp.log(w0 * 0.01)
    lw1 = jnp.log(w1 * 0.6)
    dd = lw1 - lw0
    lx = lt_ref[b, 0]
    ly = lt_ref[b, 1]
    s_y = 64.0 / dd
    t_y = (-64.0) * lw0 / dd + (32.0 * ly - 0.5)

    iy = jnp.clip(rad_ref[...] * s_y + t_y, 0.0, 63.0)
    ix = jnp.clip(tck_ref[...] + 32.0 * lx, 0.0, 63.0)
    y0f = jnp.floor(iy)
    x0f = jnp.floor(ix)
    wy = iy - y0f
    wx = ix - x0f
    y0 = y0f.astype(jnp.int32)
    x0 = x0f.astype(jnp.int32)
    y1 = jnp.minimum(y0 + 1, 63)
    x1 = jnp.minimum(x0 + 1, 63)
    i00 = y0 * 64 + x0
    i01 = y0 * 64 + x1
    i10 = y1 * 64 + x0
    i11 = y1 * 64 + x1
    q = 0.25
    w00 = (1.0 - wy) * (1.0 - wx) * q
    w01 = (1.0 - wy) * wx * q
    w10 = wy * (1.0 - wx) * q
    w11 = wy * wx * q
    idx16 = jnp.concatenate([i00, i01, i10, i11], axis=0)
    wgt16 = jnp.concatenate([w00, w01, w10, w11], axis=0)

    cy = jnp.clip(32.0 * ly + 31.5, 0.0, 63.0)
    cx = jnp.clip(32.0 * lx + 31.5, 0.0, 63.0)
    cy0f = jnp.floor(cy)
    cx0f = jnp.floor(cx)
    cwy = cy - cy0f
    cwx = cx - cx0f
    cy0 = cy0f.astype(jnp.int32)
    cx0 = cx0f.astype(jnp.int32)
    cy1 = jnp.minimum(cy0 + 1, 63)
    cx1 = jnp.minimum(cx0 + 1, 63)
    row = lax.broadcasted_iota(jnp.int32, (_NS, _NG), 0)
    col = lax.broadcasted_iota(jnp.int32, (_NS, _NG), 1)
    cidx = jnp.where(row == 0, cy0 * 64 + cx0,
           jnp.where(row == 1, cy0 * 64 + cx1,
           jnp.where(row == 2, cy1 * 64 + cx0,
           jnp.where(row == 3, cy1 * 64 + cx1, 0))))
    cwgt = jnp.where(row == 0, (1.0 - cwy) * (1.0 - cwx),
           jnp.where(row == 1, (1.0 - cwy) * cwx,
           jnp.where(row == 2, cwy * (1.0 - cwx),
           jnp.where(row == 3, cwy * cwx, 0.0))))
    idx_f = jnp.where(col < 512, idx16, jnp.where(col == 512, cidx, 0))
    wgt_f = jnp.where(col < 512, wgt16, jnp.where(col == 512, cwgt, 0.0))
    idx_f = jnp.clip(idx_f, 0, _HW - 1)
    wb = lax.bitcast_convert_type(wgt_f, jnp.int32)
    wb = (wb + 0x7FFF + (lax.shift_right_logical(wb, 16) & 1)) & ~0xFFFF
    return wb | idx_f, wcol


def _tc_prep_body(lt_ref, x_ref, f1w_ref, f1b_ref, f2w_ref, f2b_ref,
                  rad_ref, tck_ref, idx_ref, w_ref):
    pb = pl.program_id(0)
    for k in range(4):
        xb = x_ref[k].reshape(_C, 32, 128)
        packed, wcol = _tc_prep_sub(pb * 4 + k, xb, lt_ref, f1w_ref, f1b_ref,
                                    f2w_ref, f2b_ref, rad_ref, tck_ref)
        idx_ref[k] = packed
        w_ref[k] = wcol


def _tc_prep(xd, lt, f1w, f1b, f2w, f2b, interpret=False):
    rad = jnp.asarray(_RADIAL_K)
    tck = jnp.asarray(_TC_K)
    return pl.pallas_call(
        _tc_prep_body,
        grid=(_B // 4,),
        in_specs=[
            pl.BlockSpec(memory_space=pltpu.SMEM),
            pl.BlockSpec((4, _C * 32, 128), lambda b: (b, 0, 0)),
            pl.BlockSpec((48, _C), lambda b: (0, 0)),
            pl.BlockSpec((48, 1), lambda b: (0, 0)),
            pl.BlockSpec((2, 48), lambda b: (0, 0)),
            pl.BlockSpec((2, 1), lambda b: (0, 0)),
            pl.BlockSpec((4, _NG), lambda b: (0, 0)),
            pl.BlockSpec((4, _NG), lambda b: (0, 0)),
        ],
        out_specs=[
            pl.BlockSpec((4, _NS, _NG), lambda b: (b, 0, 0)),
            pl.BlockSpec((4, 2, 1), lambda b: (b, 0, 0)),
        ],
        out_shape=[
            jax.ShapeDtypeStruct((_B, _NS, _NG), jnp.int32),
            jax.ShapeDtypeStruct((_B, 2, 1), jnp.float32),
        ],
        interpret=interpret,
    )(lt, xd, f1w, f1b, f2w, f2b, rad, tck)


def _sc_gather_body(x_hbm, idx_hbm, out_hbm,
                    idx_v, plane_v, out_v, sem0, sem1):
    wid = lax.axis_index("s") * 2 + lax.axis_index("c")
    b = wid // 2
    c0 = (wid % 2) * (_GRP * _U)

    pltpu.sync_copy(idx_hbm.at[b], idx_v)

    rblk = _U * 32
    sems = [sem0, sem1]
    copies = [None, None]
    copies[0] = pltpu.make_async_copy(
        x_hbm.at[b, pl.ds(c0 * 32, rblk)], plane_v.at[pl.ds(0, rblk)], sems[0])
    copies[0].start()
    for g in range(_GRP):
        buf = g % 2
        copies[buf].wait()
        if g + 1 < _GRP:
            nbuf = 1 - buf
            copies[nbuf] = pltpu.make_async_copy(
                x_hbm.at[b, pl.ds((c0 + (g + 1) * _U) * 32, rblk)],
                plane_v.at[pl.ds(nbuf * rblk, rblk)], sems[nbuf])
            copies[nbuf].start()

        planes = [plane_v.at[pl.ds((buf * _U + u) * 32, 32), :]
                  for u in range(_U)]

        def chunk_body(i, carry, planes=planes):
            base = i * 16
            accs = [jnp.zeros((16,), jnp.float32) for _ in range(_U)]
            for s in range(_NS):
                iv = idx_v[s, pl.ds(base, 16)]
                wv = lax.bitcast_convert_type(iv & ~0xFFFF, jnp.float32)
                pv = lax.bitwise_and(iv, 0xFFFF)
                rv = lax.shift_right_logical(pv, 7)
                cv = lax.bitwise_and(pv, 127)
                for u in range(_U):
                    gv = plsc.load_gather(planes[u], [rv, cv])
                    accs[u] = accs[u] + wv * gv
            for u in range(_U):
                out_v[u, pl.ds(base, 16)] = accs[u]
            return carry

        lax.fori_loop(0, _NG // 16, chunk_body, 0)
        pltpu.sync_copy(out_v, out_hbm.at[b, pl.ds(c0 + g * _U, _U)])


def _sc_gather(xd, idx):
    f = pl.kernel(
        _sc_gather_body,
        out_type=jax.ShapeDtypeStruct((_B, _C, _NG), jnp.float32),
        mesh=plsc.VectorSubcoreMesh(core_axis_name="c", subcore_axis_name="s"),
        scratch_types=[
            pltpu.VMEM((_NS, _NG), jnp.int32),
            pltpu.VMEM((2 * _U * 32, 128), jnp.float32),
            pltpu.VMEM((_U, _NG), jnp.float32),
            pltpu.SemaphoreType.DMA,
            pltpu.SemaphoreType.DMA,
        ],
        compiler_params=pltpu.CompilerParams(needs_layout_passes=False),
    )
    return f(xd, idx)


def kernel(x, l_t_prev, fc1_w, fc1_b, fc2_w, fc2_b):
    xd = x.reshape(_B, _C * 32, 128)
    idx, wout = _tc_prep(
        xd, l_t_prev, fc1_w, fc1_b.reshape(48, 1), fc2_w, fc2_b.reshape(2, 1))
    weight = wout[:, :, 0]
    p = _sc_gather(xd, idx)
    fov = p[:, :, :512].reshape(_B, _C, 16, 32)
    vconst = p[:, :, 512]
    pooled = jnp.broadcast_to(vconst[:, :, None, None], (_B, _C, 32, 64))
    pooled = pooled.at[:, :, :16, :32].set(fov)
    return pooled, weight
